# baseline (speedup 1.0000x reference)
import dataclasses
import functools

import jax
import jax.numpy as jnp
from jax import lax
from jax.experimental import pallas as pl
from jax.experimental.pallas import tpu as pltpu
from jax.experimental.pallas import tpu_sc as plsc

NC, NS, L = 2, 16, 16
NW = NC * NS
EPS = 1e-5


def _sc_mesh():
    return plsc.VectorSubcoreMesh(core_axis_name="c", subcore_axis_name="s")


def _no_layout_params():
    cp = pltpu.CompilerParams()
    if "needs_layout_passes" in pltpu.CompilerParams.__dataclass_fields__:
        cp = dataclasses.replace(cp, needs_layout_passes=False)
    return cp


def _sc_linear_params():
    cp = pltpu.CompilerParams()
    if "use_tc_tiling_on_sc" in pltpu.CompilerParams.__dataclass_fields__:
        cp = dataclasses.replace(cp, use_tc_tiling_on_sc=False)
    return cp


def _sc_histogram(dst2, n_pad, ew):
    kg = ew // 128
    inner = 8
    outer = kg // inner
    zr = n_pad // NS

    @functools.partial(
        pl.kernel,
        out_type=jax.ShapeDtypeStruct((NC, n_pad), jnp.float32),
        mesh=_sc_mesh(),
        compiler_params=_sc_linear_params(),
        scratch_types=[
            pltpu.VMEM((inner, 128), jnp.int32),
            pltpu.VMEM((128,), jnp.float32),
            pltpu.VMEM((zr,), jnp.float32),
            pltpu.VMEM_SHARED((n_pad,), jnp.float32),
        ],
    )
    def k(dst_ref, out_ref, idx_d, ones_v, stage_v, acc_sh):
        c = lax.axis_index("c")
        s = lax.axis_index("s")
        wid = c * NS + s
        ones16 = jnp.ones((L,), jnp.float32)
        zeros16 = jnp.zeros((L,), jnp.float32)

        @pl.loop(0, 128 // L)
        def _(i):
            ones_v[pl.ds(i * L, L)] = ones16

        @pl.loop(0, zr // L)
        def _(i):
            stage_v[pl.ds(i * L, L)] = zeros16

        r0 = s * zr
        pltpu.sync_copy(stage_v, acc_sh.at[pl.ds(r0, zr)])
        plsc.subcore_barrier()

        @pl.loop(0, outer)
        def _(o):
            k0 = wid * kg + o * inner
            pltpu.sync_copy(dst_ref.at[pl.ds(k0, inner)], idx_d)
            for j in range(inner):
                pltpu.sync_copy(ones_v, acc_sh.at[idx_d.at[j]], add=True)

        plsc.subcore_barrier()
        pltpu.sync_copy(acc_sh.at[pl.ds(r0, zr)], stage_v)
        pltpu.sync_copy(stage_v, out_ref.at[c, pl.ds(r0, zr)])

    return k(dst2)


def _sc_agg16(src2, dst2, xs, n_pad, ew):
    kg = ew // 128
    ch = 4
    sup = 40
    nsup = kg // sup
    pairs = sup // (2 * ch)
    zr = n_pad // NS

    @functools.partial(
        pl.kernel,
        out_type=jax.ShapeDtypeStruct((NC, n_pad, 16), jnp.float32),
        mesh=_sc_mesh(),
        compiler_params=_sc_linear_params(),
        scratch_types=[
            pltpu.VMEM((sup, 128), jnp.int32),
            pltpu.VMEM((sup, 128), jnp.int32),
            pltpu.VMEM((2 * ch * 128, 16), jnp.float32),
            pltpu.VMEM_SHARED((n_pad, 16), jnp.float32),
            pltpu.SemaphoreType.DMA,
            pltpu.SemaphoreType.DMA,
        ],
    )
    def k(src_ref, dst_ref, xs_ref, out_ref, src_v, dst_v, rows_v, acc_sh,
          sem_a, sem_b):
        c = lax.axis_index("c")
        s = lax.axis_index("s")
        wid = c * NS + s
        e0 = wid * kg
        zeros16 = jnp.zeros((L,), jnp.float32)
        nrv = 2 * ch * 128

        @pl.loop(0, nrv)
        def _(i):
            rows_v[i, :] = zeros16

        r0 = s * zr
        for q in range(3):
            pltpu.sync_copy(rows_v, acc_sh.at[pl.ds(r0 + q * nrv, nrv)])
        pltpu.sync_copy(rows_v.at[pl.ds(0, zr - 3 * nrv)],
                        acc_sh.at[pl.ds(r0 + 3 * nrv, zr - 3 * nrv)])
        plsc.subcore_barrier()

        def descs(buf, row, sem):
            return [
                pltpu.make_async_copy(
                    xs_ref.at[src_v.at[row + j]],
                    rows_v.at[pl.ds((buf * ch + j) * 128, 128)], sem)
                for j in range(ch)
            ]

        def fire(buf, row, sem):
            for d in descs(buf, row, sem):
                d.start()

        def drain(buf, row, sem):
            for d in descs(buf, row, sem):
                d.wait()

        def scatter(buf, row):
            for j in range(ch):
                pltpu.sync_copy(rows_v.at[pl.ds((buf * ch + j) * 128, 128)],
                                acc_sh.at[dst_v.at[row + j]], add=True)

        @pl.loop(0, nsup)
        def _(u):
            g0 = e0 + u * sup
            pltpu.sync_copy(src_ref.at[pl.ds(g0, sup)], src_v)
            pltpu.sync_copy(dst_ref.at[pl.ds(g0, sup)], dst_v)
            fire(0, 0, sem_a)
            fire(1, ch, sem_b)

            @pl.loop(0, pairs - 1)
            def _(o):
                row_a = 2 * o * ch
                drain(0, row_a, sem_a)
                scatter(0, row_a)
                fire(0, row_a + 2 * ch, sem_a)
                row_b = row_a + ch
                drain(1, row_b, sem_b)
                scatter(1, row_b)
                fire(1, row_b + 2 * ch, sem_b)

            last_a = (2 * pairs - 2) * ch
            drain(0, last_a, sem_a)
            scatter(0, last_a)
            drain(1, last_a + ch, sem_b)
            scatter(1, last_a + ch)

        plsc.subcore_barrier()
        pltpu.sync_copy(acc_sh.at[pl.ds(r0, zr)],
                        out_ref.at[c, pl.ds(r0, zr)])

    return k(src2, dst2, xs)


def _sc_agg32(src2, dst2, h1s, n_pad, ew):
    kt = (NW * ew) // (128 * NS)
    ch = 2
    sup = 40
    nsup = kt // sup
    pairs = sup // (2 * ch)
    zr = n_pad // NS

    @functools.partial(
        pl.kernel,
        out_type=jax.ShapeDtypeStruct((NC, n_pad, 32), jnp.float32),
        mesh=_sc_mesh(),
        compiler_params=_sc_linear_params(),
        scratch_types=[
            pltpu.VMEM((sup, 128), jnp.int32),
            pltpu.VMEM((sup, 128), jnp.int32),
            pltpu.VMEM((2 * ch * 128, 32), jnp.float32),
            pltpu.VMEM_SHARED((n_pad, 32), jnp.float32),
            pltpu.SemaphoreType.DMA,
            pltpu.SemaphoreType.DMA,
        ],
    )
    def k(src_ref, dst_ref, tab_ref, out_ref, src_v, dst_v, rows_v, acc_sh,
          sem_a, sem_b):
        c = lax.axis_index("c")
        s = lax.axis_index("s")
        tab_c = tab_ref.at[c]
        zeros16 = jnp.zeros((L,), jnp.float32)
        nrv = 2 * ch * 128

        @pl.loop(0, nrv * 2)
        def _(i):
            rows_v[i // 2, pl.ds((i % 2) * L, L)] = zeros16

        r0 = s * zr
        for q in range(6):
            pltpu.sync_copy(rows_v, acc_sh.at[pl.ds(r0 + q * nrv, nrv)])
        pltpu.sync_copy(rows_v.at[pl.ds(0, zr - 6 * nrv)],
                        acc_sh.at[pl.ds(r0 + 6 * nrv, zr - 6 * nrv)])
        plsc.subcore_barrier()

        def descs(buf, row, sem):
            return [
                pltpu.make_async_copy(
                    tab_c.at[src_v.at[row + j]],
                    rows_v.at[pl.ds((buf * ch + j) * 128, 128)], sem)
                for j in range(ch)
            ]

        def fire(buf, row, sem):
            for d in descs(buf, row, sem):
                d.start()

        def drain(buf, row, sem):
            for d in descs(buf, row, sem):
                d.wait()

        def scatter(buf, row):
            for j in range(ch):
                pltpu.sync_copy(rows_v.at[pl.ds((buf * ch + j) * 128, 128)],
                                acc_sh.at[dst_v.at[row + j]], add=True)

        @pl.loop(0, nsup)
        def _(u):
            g0 = s * kt + u * sup
            pltpu.sync_copy(src_ref.at[pl.ds(g0, sup)], src_v)
            pltpu.sync_copy(dst_ref.at[pl.ds(g0, sup)], dst_v)
            fire(0, 0, sem_a)
            fire(1, ch, sem_b)

            @pl.loop(0, pairs - 1)
            def _(o):
                row_a = 2 * o * ch
                drain(0, row_a, sem_a)
                scatter(0, row_a)
                fire(0, row_a + 2 * ch, sem_a)
                row_b = row_a + ch
                drain(1, row_b, sem_b)
                scatter(1, row_b)
                fire(1, row_b + 2 * ch, sem_b)

            last_a = (2 * pairs - 2) * ch
            drain(0, last_a, sem_a)
            scatter(0, last_a)
            drain(1, last_a + ch, sem_b)
            scatter(1, last_a + ch)

        plsc.subcore_barrier()
        pltpu.sync_copy(acc_sh.at[pl.ds(r0, zr)],
                        out_ref.at[c, pl.ds(r0, zr)])

    return k(src2, dst2, h1s)


_BLK = 5000


def _tc_prep(hist, x16p, n_pad):
    cb = 2304
    nb = n_pad // cb

    def body(h_ref, x_ref, dinv_ref, xs_ref):
        ones = jnp.ones((NC, 1), jnp.float32)
        deg = 1.0 + lax.dot_general(h_ref[...], ones, (((0,), (0,)), ((), ())),
                                    preferred_element_type=jnp.float32,
                            precision=lax.Precision.HIGHEST)
        dinv = lax.rsqrt(deg)
        dinv_ref[...] = dinv
        xs_ref[...] = x_ref[...] * dinv

    return pl.pallas_call(
        body,
        grid=(nb,),
        in_specs=[
            pl.BlockSpec((NC, cb), lambda i: (0, i)),
            pl.BlockSpec((cb, 16), lambda i: (i, 0)),
        ],
        out_specs=[
            pl.BlockSpec((cb, 1), lambda i: (i, 0)),
            pl.BlockSpec((cb, 16), lambda i: (i, 0)),
        ],
        out_shape=[
            jax.ShapeDtypeStruct((n_pad, 1), jnp.float32),
            jax.ShapeDtypeStruct((n_pad, 16), jnp.float32),
        ],
    )(hist, x16p)


def _tc_moments16(agg1, xs, dinv, n):
    nb = n // _BLK

    def body(agg_ref, xs_ref, dinv_ref, a_ref, m_ref, cs_ref):
        i = pl.program_id(0)
        agg = agg_ref[...]
        a = (agg[0] + agg[1] + xs_ref[...]) * dinv_ref[...]
        a_ref[...] = a
        m = lax.dot_general(a, a, (((0,), (0,)), ((), ())),
                            preferred_element_type=jnp.float32,
                            precision=lax.Precision.HIGHEST)
        cs = jnp.broadcast_to(jnp.sum(a, axis=0, keepdims=True), (8, 16))

        @pl.when(i == 0)
        def _():
            m_ref[...] = m
            cs_ref[...] = cs

        @pl.when(i > 0)
        def _():
            m_ref[...] += m
            cs_ref[...] += cs

    return pl.pallas_call(
        body,
        grid=(nb,),
        in_specs=[
            pl.BlockSpec((NC, _BLK, 16), lambda i: (0, i, 0)),
            pl.BlockSpec((_BLK, 16), lambda i: (i, 0)),
            pl.BlockSpec((_BLK, 1), lambda i: (i, 0)),
        ],
        out_specs=[
            pl.BlockSpec((_BLK, 16), lambda i: (i, 0)),
            pl.BlockSpec((16, 16), lambda i: (0, 0)),
            pl.BlockSpec((8, 16), lambda i: (0, 0)),
        ],
        out_shape=[
            jax.ShapeDtypeStruct((n, 16), jnp.float32),
            jax.ShapeDtypeStruct((16, 16), jnp.float32),
            jax.ShapeDtypeStruct((8, 16), jnp.float32),
        ],
    )(agg1, xs, dinv)


def _bn_eff(m_full, cs, w, b, g, be, n):
    mean = cs[0:1, :] / n
    outer_mm = lax.dot_general(mean, mean, (((0,), (0,)), ((), ())),
                               preferred_element_type=jnp.float32,
                            precision=lax.Precision.HIGHEST)
    cov = m_full / n - outer_mm
    mz = lax.dot_general(mean, w, (((1,), (0,)), ((), ())),
                         preferred_element_type=jnp.float32,
                            precision=lax.Precision.HIGHEST) + b
    var = jnp.sum(w * lax.dot_general(cov, w, (((1,), (0,)), ((), ())),
                                      preferred_element_type=jnp.float32,
                            precision=lax.Precision.HIGHEST),
                  axis=0, keepdims=True)
    sc = g * lax.rsqrt(var + EPS)
    return w * sc, (b - mz) * sc + be


def _tc_layer1(a1, dinv, m1, cs1, w1p, b1, g1, be1, n):
    nb = n // _BLK

    def body(a_ref, dinv_ref, m_ref, cs_ref, w_ref, b_ref, g_ref, be_ref,
             out_ref):
        p = pl.program_id(0)
        weff, ceff = _bn_eff(m_ref[...], cs_ref[...], w_ref[...], b_ref[...],
                             g_ref[...], be_ref[...], n)
        h1 = jnp.maximum(
            lax.dot_general(a_ref[...], weff, (((1,), (0,)), ((), ())),
                            preferred_element_type=jnp.float32,
                            precision=lax.Precision.HIGHEST) + ceff, 0.0)
        h1s = h1 * dinv_ref[...]

        @pl.when(p == 0)
        def _():
            out_ref[...] = h1s[:, :32].reshape(1, _BLK, 32)

        @pl.when(p == 1)
        def _():
            out_ref[...] = h1s[:, 32:].reshape(1, _BLK, 32)

    return pl.pallas_call(
        body,
        grid=(2, nb),
        in_specs=[
            pl.BlockSpec((_BLK, 16), lambda p, i: (i, 0)),
            pl.BlockSpec((_BLK, 1), lambda p, i: (i, 0)),
            pl.BlockSpec((16, 16), lambda p, i: (0, 0)),
            pl.BlockSpec((8, 16), lambda p, i: (0, 0)),
            pl.BlockSpec((16, 64), lambda p, i: (0, 0)),
            pl.BlockSpec((1, 64), lambda p, i: (0, 0)),
            pl.BlockSpec((1, 64), lambda p, i: (0, 0)),
            pl.BlockSpec((1, 64), lambda p, i: (0, 0)),
        ],
        out_specs=pl.BlockSpec((1, _BLK, 32), lambda p, i: (p, i, 0)),
        out_shape=jax.ShapeDtypeStruct((2, n, 32), jnp.float32),
    )(a1, dinv, m1, cs1, w1p, b1, g1, be1)


def _tc_moments64(agg2, h1s3, dinv, n):
    blk = 2000
    nb = n // blk

    def body(agg_ref, h_ref, dinv_ref, a_ref, m_ref, cs_ref):
        i = pl.program_id(0)
        t = agg_ref[...] + h_ref[...]
        a = jnp.concatenate([t[0], t[1]], axis=-1) * dinv_ref[...]
        a_ref[...] = a
        m = lax.dot_general(a, a, (((0,), (0,)), ((), ())),
                            preferred_element_type=jnp.float32,
                            precision=lax.Precision.HIGHEST)
        cs = jnp.broadcast_to(jnp.sum(a, axis=0, keepdims=True), (8, 64))

        @pl.when(i == 0)
        def _():
            m_ref[...] = m
            cs_ref[...] = cs

        @pl.when(i > 0)
        def _():
            m_ref[...] += m
            cs_ref[...] += cs

    return pl.pallas_call(
        body,
        grid=(nb,),
        in_specs=[
            pl.BlockSpec((NC, blk, 32), lambda i: (0, i, 0)),
            pl.BlockSpec((NC, blk, 32), lambda i: (0, i, 0)),
            pl.BlockSpec((blk, 1), lambda i: (i, 0)),
        ],
        out_specs=[
            pl.BlockSpec((blk, 64), lambda i: (i, 0)),
            pl.BlockSpec((64, 64), lambda i: (0, 0)),
            pl.BlockSpec((8, 64), lambda i: (0, 0)),
        ],
        out_shape=[
            jax.ShapeDtypeStruct((n, 64), jnp.float32),
            jax.ShapeDtypeStruct((64, 64), jnp.float32),
            jax.ShapeDtypeStruct((8, 64), jnp.float32),
        ],
    )(agg2, h1s3, dinv)


def _tc_head(a2, batch3, m2, cs2, w2, b2, g2, be2, clim16, wc16, bcv, wk1a,
             wk1b, bk1v, wk2p, bk2p, n):
    nb = n // _BLK

    def body(a_ref, bt_ref, m_ref, cs_ref, w_ref, b_ref, g_ref, be_ref,
             cl_ref, wc_ref, bc_ref, wa_ref, wb_ref, bk1_ref, wk2_ref,
             bk2_ref, out_ref):
        weff, ceff = _bn_eff(m_ref[...], cs_ref[...], w_ref[...], b_ref[...],
                             g_ref[...], be_ref[...], n)
        h2 = jnp.maximum(
            lax.dot_general(a_ref[...], weff, (((1,), (0,)), ((), ())),
                            preferred_element_type=jnp.float32,
                            precision=lax.Precision.HIGHEST) + ceff, 0.0)
        ce = jnp.maximum(
            lax.dot_general(cl_ref[...], wc_ref[...], (((1,), (0,)), ((), ())),
                            preferred_element_type=jnp.float32,
                            precision=lax.Precision.HIGHEST) + bc_ref[...],
            0.0)
        cx = lax.dot_general(ce, wb_ref[...], (((1,), (0,)), ((), ())),
                             preferred_element_type=jnp.float32,
                            precision=lax.Precision.HIGHEST)
        bvec = bt_ref[0]
        io = lax.broadcasted_iota(jnp.int32, (128, 1), 0)
        oh = (io == bvec).astype(jnp.float32)
        cxg = lax.dot_general(oh, cx, (((0,), (0,)), ((), ())),
                              preferred_element_type=jnp.float32,
                            precision=lax.Precision.HIGHEST)
        t = jnp.maximum(
            lax.dot_general(h2, wa_ref[...], (((1,), (0,)), ((), ())),
                            preferred_element_type=jnp.float32,
                            precision=lax.Precision.HIGHEST)
            + cxg + bk1_ref[...], 0.0)
        out_ref[...] = lax.dot_general(t, wk2_ref[...], (((1,), (0,)), ((), ())),
                                       preferred_element_type=jnp.float32,
                            precision=lax.Precision.HIGHEST) \
            + bk2_ref[...]

    return pl.pallas_call(
        body,
        grid=(nb,),
        in_specs=[
            pl.BlockSpec((_BLK, 64), lambda i: (i, 0)),
            pl.BlockSpec((1, 1, _BLK), lambda i: (i, 0, 0)),
            pl.BlockSpec((64, 64), lambda i: (0, 0)),
            pl.BlockSpec((8, 64), lambda i: (0, 0)),
            pl.BlockSpec((64, 64), lambda i: (0, 0)),
            pl.BlockSpec((1, 64), lambda i: (0, 0)),
            pl.BlockSpec((1, 64), lambda i: (0, 0)),
            pl.BlockSpec((1, 64), lambda i: (0, 0)),
            pl.BlockSpec((128, 16), lambda i: (0, 0)),
            pl.BlockSpec((16, 64), lambda i: (0, 0)),
            pl.BlockSpec((1, 64), lambda i: (0, 0)),
            pl.BlockSpec((64, 64), lambda i: (0, 0)),
            pl.BlockSpec((64, 64), lambda i: (0, 0)),
            pl.BlockSpec((1, 64), lambda i: (0, 0)),
            pl.BlockSpec((64, 8), lambda i: (0, 0)),
            pl.BlockSpec((1, 8), lambda i: (0, 0)),
        ],
        out_specs=pl.BlockSpec((_BLK, 8), lambda i: (i, 0)),
        out_shape=jax.ShapeDtypeStruct((n, 8), jnp.float32),
    )(a2, batch3, m2, cs2, w2, b2, g2, be2, clim16, wc16, bcv, wk1a, wk1b,
      bk1v, wk2p, bk2p)


def kernel(x, edge_index, batch, climber, W1, b1, g1, be1, W2, b2, g2, be2,
           Wc, bc, Wk1, bk1, Wk2, bk2):
    n = x.shape[0]
    e = edge_index.shape[1]
    n_pad = -(-(n + 16) // 768) * 768
    ew = (-(-e // NW) + 1023) // 1024 * 1024
    ep = NW * ew

    src = edge_index[0].astype(jnp.int32)
    dst = edge_index[1].astype(jnp.int32)
    padn = ep - e
    pad_ar = jnp.arange(padn, dtype=jnp.int32)
    src_p = jnp.concatenate([src, pad_ar % 256])
    dst_p = jnp.concatenate([dst, n + (pad_ar % 16)])
    src2 = src_p.reshape(ep // 128, 128)
    dst2 = dst_p.reshape(ep // 128, 128)

    x16p = jnp.pad(x, ((0, n_pad - n), (0, 16 - x.shape[1])))
    w1p = jnp.pad(W1, ((0, 16 - W1.shape[0]), (0, 0)))
    clim16 = jnp.pad(climber, ((0, 0), (0, 16 - climber.shape[1])))
    wc16 = jnp.pad(Wc, ((0, 16 - Wc.shape[0]), (0, 0)))
    wk1a, wk1b = Wk1[:64], Wk1[64:]
    wk2p = jnp.pad(Wk2, ((0, 0), (0, 8 - Wk2.shape[1])))
    bk2p = jnp.pad(bk2, (0, 8 - bk2.shape[0])).reshape(1, 8)
    b1v, g1v, be1v = b1.reshape(1, 64), g1.reshape(1, 64), be1.reshape(1, 64)
    b2v, g2v, be2v = b2.reshape(1, 64), g2.reshape(1, 64), be2.reshape(1, 64)
    bcv, bk1v = bc.reshape(1, 64), bk1.reshape(1, 64)
    batch3 = batch.astype(jnp.int32).reshape(n // _BLK, 1, _BLK)

    hist = _sc_histogram(dst2, n_pad, ew)
    dinv, xs = _tc_prep(hist, x16p, n_pad)
    agg1 = _sc_agg16(src2, dst2, xs, n_pad, ew)
    a1, m1, cs1 = _tc_moments16(agg1, xs, dinv, n)
    h1s = _tc_layer1(a1, dinv, m1, cs1, w1p, b1v, g1v, be1v, n)
    agg2 = _sc_agg32(src2, dst2, h1s, n_pad, ew)
    a2, m2, cs2 = _tc_moments64(agg2, h1s, dinv, n)
    out8 = _tc_head(a2, batch3, m2, cs2, W2, b2v, g2v, be2v, clim16, wc16,
                    bcv, wk1a, wk1b, bk1v, wk2p, bk2p, n)
    return out8[:, :4]


# speedup vs baseline: 1.1663x; 1.1663x over previous
import dataclasses
import functools

import jax
import jax.numpy as jnp
from jax import lax
from jax.experimental import pallas as pl
from jax.experimental.pallas import tpu as pltpu
from jax.experimental.pallas import tpu_sc as plsc

NC, NS, L = 2, 16, 16
NW = NC * NS
EPS = 1e-5


def _sc_mesh():
    return plsc.VectorSubcoreMesh(core_axis_name="c", subcore_axis_name="s")


def _no_layout_params():
    cp = pltpu.CompilerParams()
    if "needs_layout_passes" in pltpu.CompilerParams.__dataclass_fields__:
        cp = dataclasses.replace(cp, needs_layout_passes=False)
    return cp


def _sc_linear_params():
    cp = pltpu.CompilerParams()
    if "use_tc_tiling_on_sc" in pltpu.CompilerParams.__dataclass_fields__:
        cp = dataclasses.replace(cp, use_tc_tiling_on_sc=False)
    return cp


def _sc_histogram(dst2, n_pad, ew):
    kg = ew // 128
    inner = 8
    outer = kg // inner
    zr = n_pad // NS

    @functools.partial(
        pl.kernel,
        out_type=jax.ShapeDtypeStruct((NC, n_pad), jnp.float32),
        mesh=_sc_mesh(),
        compiler_params=_sc_linear_params(),
        scratch_types=[
            pltpu.VMEM((inner, 128), jnp.int32),
            pltpu.VMEM((128,), jnp.float32),
            pltpu.VMEM((zr,), jnp.float32),
            pltpu.VMEM_SHARED((n_pad,), jnp.float32),
        ],
    )
    def k(dst_ref, out_ref, idx_d, ones_v, stage_v, acc_sh):
        c = lax.axis_index("c")
        s = lax.axis_index("s")
        wid = c * NS + s
        ones16 = jnp.ones((L,), jnp.float32)
        zeros16 = jnp.zeros((L,), jnp.float32)

        @pl.loop(0, 128 // L)
        def _(i):
            ones_v[pl.ds(i * L, L)] = ones16

        @pl.loop(0, zr // L)
        def _(i):
            stage_v[pl.ds(i * L, L)] = zeros16

        r0 = s * zr
        pltpu.sync_copy(stage_v, acc_sh.at[pl.ds(r0, zr)])
        plsc.subcore_barrier()

        @pl.loop(0, outer)
        def _(o):
            k0 = wid * kg + o * inner
            pltpu.sync_copy(dst_ref.at[pl.ds(k0, inner)], idx_d)
            for j in range(inner):
                pltpu.sync_copy(ones_v, acc_sh.at[idx_d.at[j]], add=True)

        plsc.subcore_barrier()
        pltpu.sync_copy(acc_sh.at[pl.ds(r0, zr)], stage_v)
        pltpu.sync_copy(stage_v, out_ref.at[c, pl.ds(r0, zr)])

    return k(dst2)


def _sc_agg16(src2, dst2, xs, n_pad, ew):
    kg = ew // 128
    ch = 4
    sup = 40
    nsup = kg // sup
    pairs = sup // (2 * ch)
    zr = n_pad // NS

    @functools.partial(
        pl.kernel,
        out_type=jax.ShapeDtypeStruct((NC, n_pad, 16), jnp.float32),
        mesh=_sc_mesh(),
        compiler_params=_sc_linear_params(),
        scratch_types=[
            pltpu.VMEM((sup, 128), jnp.int32),
            pltpu.VMEM((sup, 128), jnp.int32),
            pltpu.VMEM((2 * ch * 128, 16), jnp.float32),
            pltpu.VMEM_SHARED((n_pad, 16), jnp.float32),
            pltpu.SemaphoreType.DMA,
            pltpu.SemaphoreType.DMA,
        ],
    )
    def k(src_ref, dst_ref, xs_ref, out_ref, src_v, dst_v, rows_v, acc_sh,
          sem_a, sem_b):
        c = lax.axis_index("c")
        s = lax.axis_index("s")
        wid = c * NS + s
        e0 = wid * kg
        zeros16 = jnp.zeros((L,), jnp.float32)
        nrv = 2 * ch * 128

        @pl.loop(0, nrv)
        def _(i):
            rows_v[i, :] = zeros16

        r0 = s * zr
        for q in range(3):
            pltpu.sync_copy(rows_v, acc_sh.at[pl.ds(r0 + q * nrv, nrv)])
        pltpu.sync_copy(rows_v.at[pl.ds(0, zr - 3 * nrv)],
                        acc_sh.at[pl.ds(r0 + 3 * nrv, zr - 3 * nrv)])
        plsc.subcore_barrier()

        def descs(buf, row, sem):
            return [
                pltpu.make_async_copy(
                    xs_ref.at[src_v.at[row + j]],
                    rows_v.at[pl.ds((buf * ch + j) * 128, 128)], sem)
                for j in range(ch)
            ]

        def fire(buf, row, sem):
            for d in descs(buf, row, sem):
                d.start()

        def drain(buf, row, sem):
            for d in descs(buf, row, sem):
                d.wait()

        def scatter(buf, row):
            for j in range(ch):
                pltpu.sync_copy(rows_v.at[pl.ds((buf * ch + j) * 128, 128)],
                                acc_sh.at[dst_v.at[row + j]], add=True)

        @pl.loop(0, nsup)
        def _(u):
            g0 = e0 + u * sup
            pltpu.sync_copy(src_ref.at[pl.ds(g0, sup)], src_v)
            pltpu.sync_copy(dst_ref.at[pl.ds(g0, sup)], dst_v)
            fire(0, 0, sem_a)
            fire(1, ch, sem_b)

            @pl.loop(0, pairs - 1)
            def _(o):
                row_a = 2 * o * ch
                drain(0, row_a, sem_a)
                scatter(0, row_a)
                fire(0, row_a + 2 * ch, sem_a)
                row_b = row_a + ch
                drain(1, row_b, sem_b)
                scatter(1, row_b)
                fire(1, row_b + 2 * ch, sem_b)

            last_a = (2 * pairs - 2) * ch
            drain(0, last_a, sem_a)
            scatter(0, last_a)
            drain(1, last_a + ch, sem_b)
            scatter(1, last_a + ch)

        plsc.subcore_barrier()
        pltpu.sync_copy(acc_sh.at[pl.ds(r0, zr)],
                        out_ref.at[c, pl.ds(r0, zr)])

    return k(src2, dst2, xs)


def _sc_agg32(src2, dst2, h1s, n_pad, ew):
    kt = (NW * ew) // (128 * NS)
    ch = 2
    sup = 40
    nsup = kt // sup
    pairs = sup // (2 * ch)
    zr = n_pad // NS

    @functools.partial(
        pl.kernel,
        out_type=jax.ShapeDtypeStruct((NC, n_pad, 32), jnp.float32),
        mesh=_sc_mesh(),
        compiler_params=_sc_linear_params(),
        scratch_types=[
            pltpu.VMEM((sup, 128), jnp.int32),
            pltpu.VMEM((sup, 128), jnp.int32),
            pltpu.VMEM((2 * ch * 128, 32), jnp.float32),
            pltpu.VMEM_SHARED((n_pad, 32), jnp.float32),
            pltpu.SemaphoreType.DMA,
            pltpu.SemaphoreType.DMA,
        ],
    )
    def k(src_ref, dst_ref, tab_ref, out_ref, src_v, dst_v, rows_v, acc_sh,
          sem_a, sem_b):
        c = lax.axis_index("c")
        s = lax.axis_index("s")
        tab_c = tab_ref.at[c]
        zeros16 = jnp.zeros((L,), jnp.float32)
        nrv = 2 * ch * 128

        @pl.loop(0, nrv * 2)
        def _(i):
            rows_v[i // 2, pl.ds((i % 2) * L, L)] = zeros16

        r0 = s * zr
        for q in range(6):
            pltpu.sync_copy(rows_v, acc_sh.at[pl.ds(r0 + q * nrv, nrv)])
        pltpu.sync_copy(rows_v.at[pl.ds(0, zr - 6 * nrv)],
                        acc_sh.at[pl.ds(r0 + 6 * nrv, zr - 6 * nrv)])
        plsc.subcore_barrier()

        def descs(buf, row, sem):
            return [
                pltpu.make_async_copy(
                    tab_c.at[src_v.at[row + j]],
                    rows_v.at[pl.ds((buf * ch + j) * 128, 128)], sem)
                for j in range(ch)
            ]

        def fire(buf, row, sem):
            for d in descs(buf, row, sem):
                d.start()

        def drain(buf, row, sem):
            for d in descs(buf, row, sem):
                d.wait()

        def scatter(buf, row):
            for j in range(ch):
                pltpu.sync_copy(rows_v.at[pl.ds((buf * ch + j) * 128, 128)],
                                acc_sh.at[dst_v.at[row + j]], add=True)

        @pl.loop(0, nsup)
        def _(u):
            g0 = s * kt + u * sup
            pltpu.sync_copy(src_ref.at[pl.ds(g0, sup)], src_v)
            pltpu.sync_copy(dst_ref.at[pl.ds(g0, sup)], dst_v)
            fire(0, 0, sem_a)
            fire(1, ch, sem_b)

            @pl.loop(0, pairs - 1)
            def _(o):
                row_a = 2 * o * ch
                drain(0, row_a, sem_a)
                scatter(0, row_a)
                fire(0, row_a + 2 * ch, sem_a)
                row_b = row_a + ch
                drain(1, row_b, sem_b)
                scatter(1, row_b)
                fire(1, row_b + 2 * ch, sem_b)

            last_a = (2 * pairs - 2) * ch
            drain(0, last_a, sem_a)
            scatter(0, last_a)
            drain(1, last_a + ch, sem_b)
            scatter(1, last_a + ch)

        plsc.subcore_barrier()
        pltpu.sync_copy(acc_sh.at[pl.ds(r0, zr)],
                        out_ref.at[c, pl.ds(r0, zr)])

    return k(src2, dst2, h1s)


_BLK = 2000


def _tc_prep(hist, x16p, n_pad):
    cb = 2304
    nb = n_pad // cb

    def body(h_ref, x_ref, dinv_ref, xs_ref):
        ones = jnp.ones((NC, 1), jnp.float32)
        deg = 1.0 + lax.dot_general(h_ref[...], ones, (((0,), (0,)), ((), ())),
                                    preferred_element_type=jnp.float32,
                            precision=lax.Precision.HIGHEST)
        dinv = lax.rsqrt(deg)
        dinv_ref[...] = dinv
        xs_ref[...] = x_ref[...] * dinv

    return pl.pallas_call(
        body,
        grid=(nb,),
        in_specs=[
            pl.BlockSpec((NC, cb), lambda i: (0, i)),
            pl.BlockSpec((cb, 16), lambda i: (i, 0)),
        ],
        out_specs=[
            pl.BlockSpec((cb, 1), lambda i: (i, 0)),
            pl.BlockSpec((cb, 16), lambda i: (i, 0)),
        ],
        out_shape=[
            jax.ShapeDtypeStruct((n_pad, 1), jnp.float32),
            jax.ShapeDtypeStruct((n_pad, 16), jnp.float32),
        ],
    )(hist, x16p)


def _tc_moments16(agg1, xs, dinv, w1p, b1, g1, be1, n):
    nb = n // _BLK

    def body(agg_ref, xs_ref, dinv_ref, w_ref, b_ref, g_ref, be_ref,
             a_ref, weff_ref, ceff_ref, m_s, cs_s):
        i = pl.program_id(0)
        agg = agg_ref[...]
        a = (agg[0] + agg[1] + xs_ref[...]) * dinv_ref[...]
        a_ref[...] = a
        m = lax.dot_general(a, a, (((0,), (0,)), ((), ())),
                            preferred_element_type=jnp.float32,
                            precision=lax.Precision.HIGHEST)
        cs = jnp.broadcast_to(jnp.sum(a, axis=0, keepdims=True), (8, 16))

        @pl.when(i == 0)
        def _():
            m_s[...] = m
            cs_s[...] = cs

        @pl.when(i > 0)
        def _():
            m_s[...] += m
            cs_s[...] += cs

        @pl.when(i == nb - 1)
        def _():
            weff, ceff = _bn_eff(m_s[...], cs_s[...], w_ref[...], b_ref[...],
                                 g_ref[...], be_ref[...], n)
            weff_ref[...] = weff
            ceff_ref[...] = jnp.broadcast_to(ceff, (8, 64))

    return pl.pallas_call(
        body,
        grid=(nb,),
        in_specs=[
            pl.BlockSpec((NC, _BLK, 16), lambda i: (0, i, 0)),
            pl.BlockSpec((_BLK, 16), lambda i: (i, 0)),
            pl.BlockSpec((_BLK, 1), lambda i: (i, 0)),
            pl.BlockSpec((16, 64), lambda i: (0, 0)),
            pl.BlockSpec((1, 64), lambda i: (0, 0)),
            pl.BlockSpec((1, 64), lambda i: (0, 0)),
            pl.BlockSpec((1, 64), lambda i: (0, 0)),
        ],
        out_specs=[
            pl.BlockSpec((_BLK, 16), lambda i: (i, 0)),
            pl.BlockSpec((16, 64), lambda i: (0, 0)),
            pl.BlockSpec((8, 64), lambda i: (0, 0)),
        ],
        out_shape=[
            jax.ShapeDtypeStruct((n, 16), jnp.float32),
            jax.ShapeDtypeStruct((16, 64), jnp.float32),
            jax.ShapeDtypeStruct((8, 64), jnp.float32),
        ],
        scratch_shapes=[
            pltpu.VMEM((16, 16), jnp.float32),
            pltpu.VMEM((8, 16), jnp.float32),
        ],
    )(agg1, xs, dinv, w1p, b1, g1, be1)


def _bn_eff(m_full, cs, w, b, g, be, n):
    mean = cs[0:1, :] / n
    outer_mm = lax.dot_general(mean, mean, (((0,), (0,)), ((), ())),
                               preferred_element_type=jnp.float32,
                            precision=lax.Precision.HIGHEST)
    cov = m_full / n - outer_mm
    mz = lax.dot_general(mean, w, (((1,), (0,)), ((), ())),
                         preferred_element_type=jnp.float32,
                            precision=lax.Precision.HIGHEST) + b
    var = jnp.sum(w * lax.dot_general(cov, w, (((1,), (0,)), ((), ())),
                                      preferred_element_type=jnp.float32,
                            precision=lax.Precision.HIGHEST),
                  axis=0, keepdims=True)
    sc = g * lax.rsqrt(var + EPS)
    return w * sc, (b - mz) * sc + be


def _tc_layer1(a1, dinv, weff1, ceff1, n):
    nb = n // _BLK

    def body(a_ref, dinv_ref, w_ref, cef_ref, out_ref):
        p = pl.program_id(0)
        h1 = jnp.maximum(
            lax.dot_general(a_ref[...], w_ref[...], (((1,), (0,)), ((), ())),
                            preferred_element_type=jnp.float32,
                            precision=lax.Precision.HIGHEST)
            + cef_ref[0:1, :], 0.0)
        h1s = h1 * dinv_ref[...]

        @pl.when(p == 0)
        def _():
            out_ref[...] = h1s[:, :32].reshape(1, _BLK, 32)

        @pl.when(p == 1)
        def _():
            out_ref[...] = h1s[:, 32:].reshape(1, _BLK, 32)

    return pl.pallas_call(
        body,
        grid=(2, nb),
        in_specs=[
            pl.BlockSpec((_BLK, 16), lambda p, i: (i, 0)),
            pl.BlockSpec((_BLK, 1), lambda p, i: (i, 0)),
            pl.BlockSpec((16, 64), lambda p, i: (0, 0)),
            pl.BlockSpec((8, 64), lambda p, i: (0, 0)),
        ],
        out_specs=pl.BlockSpec((1, _BLK, 32), lambda p, i: (p, i, 0)),
        out_shape=jax.ShapeDtypeStruct((2, n, 32), jnp.float32),
    )(a1, dinv, weff1, ceff1)


def _tc_moments64(agg2, h1s3, dinv, w2, b2, g2, be2, clim16, wc16, bcv,
                  wk1b, n):
    blk = 2000
    nb = n // blk

    def body(agg_ref, h_ref, dinv_ref, w_ref, b_ref, g_ref, be_ref, cl_ref,
             wc_ref, bc_ref, wb_ref, a_ref, weff_ref, ceff_ref, cx_ref,
             m_s, cs_s):
        i = pl.program_id(0)
        t = agg_ref[...] + h_ref[...]
        a = jnp.concatenate([t[0], t[1]], axis=-1) * dinv_ref[...]
        a_ref[...] = a
        m = lax.dot_general(a, a, (((0,), (0,)), ((), ())),
                            preferred_element_type=jnp.float32,
                            precision=lax.Precision.HIGHEST)
        cs = jnp.broadcast_to(jnp.sum(a, axis=0, keepdims=True), (8, 64))

        @pl.when(i == 0)
        def _():
            m_s[...] = m
            cs_s[...] = cs

        @pl.when(i > 0)
        def _():
            m_s[...] += m
            cs_s[...] += cs

        @pl.when(i == nb - 1)
        def _():
            weff, ceff = _bn_eff(m_s[...], cs_s[...], w_ref[...], b_ref[...],
                                 g_ref[...], be_ref[...], n)
            weff_ref[...] = weff
            ceff_ref[...] = jnp.broadcast_to(ceff, (8, 64))
            ce = jnp.maximum(
                lax.dot_general(cl_ref[...], wc_ref[...],
                                (((1,), (0,)), ((), ())),
                                preferred_element_type=jnp.float32,
                                precision=lax.Precision.HIGHEST)
                + bc_ref[...], 0.0)
            cx_ref[...] = lax.dot_general(ce, wb_ref[...],
                                          (((1,), (0,)), ((), ())),
                                          preferred_element_type=jnp.float32,
                                          precision=lax.Precision.HIGHEST)

    return pl.pallas_call(
        body,
        grid=(nb,),
        in_specs=[
            pl.BlockSpec((NC, blk, 32), lambda i: (0, i, 0)),
            pl.BlockSpec((NC, blk, 32), lambda i: (0, i, 0)),
            pl.BlockSpec((blk, 1), lambda i: (i, 0)),
            pl.BlockSpec((64, 64), lambda i: (0, 0)),
            pl.BlockSpec((1, 64), lambda i: (0, 0)),
            pl.BlockSpec((1, 64), lambda i: (0, 0)),
            pl.BlockSpec((1, 64), lambda i: (0, 0)),
            pl.BlockSpec((128, 16), lambda i: (0, 0)),
            pl.BlockSpec((16, 64), lambda i: (0, 0)),
            pl.BlockSpec((1, 64), lambda i: (0, 0)),
            pl.BlockSpec((64, 64), lambda i: (0, 0)),
        ],
        out_specs=[
            pl.BlockSpec((blk, 64), lambda i: (i, 0)),
            pl.BlockSpec((64, 64), lambda i: (0, 0)),
            pl.BlockSpec((8, 64), lambda i: (0, 0)),
            pl.BlockSpec((128, 64), lambda i: (0, 0)),
        ],
        out_shape=[
            jax.ShapeDtypeStruct((n, 64), jnp.float32),
            jax.ShapeDtypeStruct((64, 64), jnp.float32),
            jax.ShapeDtypeStruct((8, 64), jnp.float32),
            jax.ShapeDtypeStruct((128, 64), jnp.float32),
        ],
        scratch_shapes=[
            pltpu.VMEM((64, 64), jnp.float32),
            pltpu.VMEM((8, 64), jnp.float32),
        ],
    )(agg2, h1s3, dinv, w2, b2, g2, be2, clim16, wc16, bcv, wk1b)


def _tc_head(a2, batch3, weff2, ceff2, cx, wk1a, bk1v, wk2p, bk2p, n):
    nb = n // _BLK

    def body(a_ref, bt_ref, w_ref, cef_ref, cx_ref, wa_ref, bk1_ref,
             wk2_ref, bk2_ref, out_ref):
        h2 = jnp.maximum(
            lax.dot_general(a_ref[...], w_ref[...], (((1,), (0,)), ((), ())),
                            preferred_element_type=jnp.float32,
                            precision=lax.Precision.HIGHEST)
            + cef_ref[0:1, :], 0.0)
        bvec = bt_ref[0]
        io = lax.broadcasted_iota(jnp.int32, (128, 1), 0)
        oh = (io == bvec).astype(jnp.float32)
        cxg = lax.dot_general(oh, cx_ref[...], (((0,), (0,)), ((), ())),
                              preferred_element_type=jnp.float32,
                              precision=lax.Precision.HIGHEST)
        t = jnp.maximum(
            lax.dot_general(h2, wa_ref[...], (((1,), (0,)), ((), ())),
                            preferred_element_type=jnp.float32,
                            precision=lax.Precision.HIGHEST)
            + cxg + bk1_ref[...], 0.0)
        out_ref[...] = lax.dot_general(t, wk2_ref[...], (((1,), (0,)), ((), ())),
                                       preferred_element_type=jnp.float32,
                                       precision=lax.Precision.HIGHEST) \
            + bk2_ref[...]

    return pl.pallas_call(
        body,
        grid=(nb,),
        in_specs=[
            pl.BlockSpec((_BLK, 64), lambda i: (i, 0)),
            pl.BlockSpec((1, 1, _BLK), lambda i: (i, 0, 0)),
            pl.BlockSpec((64, 64), lambda i: (0, 0)),
            pl.BlockSpec((8, 64), lambda i: (0, 0)),
            pl.BlockSpec((128, 64), lambda i: (0, 0)),
            pl.BlockSpec((64, 64), lambda i: (0, 0)),
            pl.BlockSpec((1, 64), lambda i: (0, 0)),
            pl.BlockSpec((64, 8), lambda i: (0, 0)),
            pl.BlockSpec((1, 8), lambda i: (0, 0)),
        ],
        out_specs=pl.BlockSpec((_BLK, 8), lambda i: (i, 0)),
        out_shape=jax.ShapeDtypeStruct((n, 8), jnp.float32),
    )(a2, batch3, weff2, ceff2, cx, wk1a, bk1v, wk2p, bk2p)


def kernel(x, edge_index, batch, climber, W1, b1, g1, be1, W2, b2, g2, be2,
           Wc, bc, Wk1, bk1, Wk2, bk2):
    n = x.shape[0]
    e = edge_index.shape[1]
    n_pad = -(-(n + 16) // 768) * 768
    ew = (-(-e // NW) + 1023) // 1024 * 1024
    ep = NW * ew

    src = edge_index[0].astype(jnp.int32)
    dst = edge_index[1].astype(jnp.int32)
    padn = ep - e
    pad_ar = jnp.arange(padn, dtype=jnp.int32)
    src_p = jnp.concatenate([src, pad_ar % 256])
    dst_p = jnp.concatenate([dst, n + (pad_ar % 16)])
    src2 = src_p.reshape(ep // 128, 128)
    dst2 = dst_p.reshape(ep // 128, 128)

    x16p = jnp.pad(x, ((0, n_pad - n), (0, 16 - x.shape[1])))
    w1p = jnp.pad(W1, ((0, 16 - W1.shape[0]), (0, 0)))
    clim16 = jnp.pad(climber, ((0, 0), (0, 16 - climber.shape[1])))
    wc16 = jnp.pad(Wc, ((0, 16 - Wc.shape[0]), (0, 0)))
    wk1a, wk1b = Wk1[:64], Wk1[64:]
    wk2p = jnp.pad(Wk2, ((0, 0), (0, 8 - Wk2.shape[1])))
    bk2p = jnp.pad(bk2, (0, 8 - bk2.shape[0])).reshape(1, 8)
    b1v, g1v, be1v = b1.reshape(1, 64), g1.reshape(1, 64), be1.reshape(1, 64)
    b2v, g2v, be2v = b2.reshape(1, 64), g2.reshape(1, 64), be2.reshape(1, 64)
    bcv, bk1v = bc.reshape(1, 64), bk1.reshape(1, 64)
    batch3 = batch.astype(jnp.int32).reshape(n // _BLK, 1, _BLK)

    hist = _sc_histogram(dst2, n_pad, ew)
    dinv, xs = _tc_prep(hist, x16p, n_pad)
    agg1 = _sc_agg16(src2, dst2, xs, n_pad, ew)
    a1, weff1, ceff1 = _tc_moments16(agg1, xs, dinv, w1p, b1v, g1v, be1v, n)
    h1s = _tc_layer1(a1, dinv, weff1, ceff1, n)
    agg2 = _sc_agg32(src2, dst2, h1s, n_pad, ew)
    a2, weff2, ceff2, cx = _tc_moments64(agg2, h1s, dinv, W2, b2v, g2v, be2v,
                                         clim16, wc16, bcv, wk1b, n)
    out8 = _tc_head(a2, batch3, weff2, ceff2, cx, wk1a, bk1v, wk2p, bk2p, n)
    return out8[:, :4]
